# Initial kernel scaffold; baseline (speedup 1.0000x reference)
import jax
import jax.numpy as jnp
from jax.experimental import pallas as pl


def kernel(edge_logits, stop_logits, edge_scores, u, edge_batch):
    raise NotImplementedError("write your pallas kernel here")


# speedup vs baseline: 1.3603x; 1.3603x over previous
import functools

import jax
import jax.numpy as jnp
from jax.experimental import pallas as pl

LARGE_NEG = -1e9
RAP = 0.1
PRIOR_EPS = 1e-4
PROB_EPS = 1e-12


def _elem_body(el_ref, es_ref, u_ref, p_ref, w_ref):
    el = el_ref[...]
    es = es_ref[...]
    u = u_ref[...]
    p = jnp.exp(el) * jnp.sqrt(jnp.clip(es, PRIOR_EPS, None))
    uc = jnp.clip(u, 1e-9, 1.0 - 1e-9)
    w = -1.0 / jnp.log(uc)
    p_ref[...] = p
    w_ref[...] = w


def _elementwise(edge_logits, edge_scores, u):
    E = edge_logits.shape[0]
    rows = E // 128
    grid = 16
    blk = rows // grid
    el2 = edge_logits.reshape(rows, 128)
    es2 = edge_scores.reshape(rows, 128)
    u2 = u.reshape(rows, 128)
    spec = pl.BlockSpec((blk, 128), lambda i: (i, 0))
    p2, w2 = pl.pallas_call(
        _elem_body,
        grid=(grid,),
        in_specs=[spec, spec, spec],
        out_specs=[spec, spec],
        out_shape=[jax.ShapeDtypeStruct((rows, 128), jnp.float32)] * 2,
    )(el2, es2, u2)
    return p2.reshape(E), w2.reshape(E)


def kernel(edge_logits, stop_logits, edge_scores, u, edge_batch):
    E = edge_logits.shape[0]
    B = stop_logits.shape[0]
    p, w = _elementwise(edge_logits, edge_scores, u)

    ids = edge_batch
    Z = jax.ops.segment_sum(p, ids, num_segments=B)
    cnt = jax.ops.segment_sum(jnp.ones((E,), jnp.float32), ids, num_segments=B)
    exp_stop = jnp.exp(stop_logits)
    D = Z + exp_stop
    dinv = 1.0 / D
    it = 1.0 / (cnt + 1.0)

    q = (1.0 - RAP) * p * dinv[ids] + RAP * it[ids]
    s = q * w
    bestv = jax.ops.segment_max(s, ids, num_segments=B)
    has_edge = cnt > 0
    bestv = jnp.where(has_edge, bestv, 0.0)
    is_best = s >= bestv[ids]
    idx_cand = jnp.where(is_best, jnp.arange(E), E)
    best_idx = jnp.clip(jax.ops.segment_min(idx_cand, ids, num_segments=B), 0, E - 1)
    q_chosen = q[best_idx]

    final_stop = (1.0 - RAP) * exp_stop * dinv + RAP * it
    log_stop = jnp.log(jnp.clip(final_stop, PROB_EPS, None))
    log_edge = jnp.log(jnp.clip(q_chosen, PROB_EPS, None))
    stop_chosen = final_stop >= bestv
    return jnp.where(stop_chosen, log_stop, log_edge)


# speedup vs baseline: 200.4355x; 147.3456x over previous
import functools

import jax
import jax.numpy as jnp
from jax import lax
from jax.experimental import pallas as pl
from jax.experimental.pallas import tpu as pltpu
from jax.experimental.pallas import tpu_sc as plsc

RAP = 0.1
PRIOR_EPS = 1e-4
PROB_EPS = 1e-12

E = 1048576
B = 4096
NC, NS, L = 2, 16, 16
NW = NC * NS
C_PER = E // NW
ROWS_PER = C_PER // 128
S1_SUB_ROWS = 64
S2_SUB = 16384
_mesh = plsc.VectorSubcoreMesh(core_axis_name="c", subcore_axis_name="s")
_sc_params = pltpu.CompilerParams(needs_layout_passes=False)


def _e1_body(el_ref, es_ref, u_ref, p_ref, w_ref):
    p_ref[...] = jnp.exp(el_ref[...]) * jnp.sqrt(
        jnp.clip(es_ref[...], PRIOR_EPS, None))
    uc = jnp.clip(u_ref[...], 1e-9, 1.0 - 1e-9)
    w_ref[...] = -1.0 / jnp.log(uc)


def _e1(edge_logits, edge_scores, u):
    rows = E // 128
    grid = 16
    blk = rows // grid
    spec = pl.BlockSpec((blk, 128), lambda i: (i, 0))
    return pl.pallas_call(
        _e1_body,
        grid=(grid,),
        in_specs=[spec] * 3,
        out_specs=[spec] * 2,
        out_shape=[jax.ShapeDtypeStruct((rows, 128), jnp.float32)] * 2,
    )(edge_logits.reshape(rows, 128), edge_scores.reshape(rows, 128),
      u.reshape(rows, 128))


def _fill(ref, n, value):
    v = jnp.full((L,), value, dtype=ref.dtype)

    def body(i, _):
        ref[pl.ds(i * L, L)] = v
        return 0

    lax.fori_loop(0, n // L, body, 0)


@functools.partial(
    pl.kernel,
    mesh=_mesh,
    out_type=[jax.ShapeDtypeStruct((NC, B), jnp.float32),
              jax.ShapeDtypeStruct((NC, B), jnp.float32)],
    scratch_types=[
        pltpu.VMEM((S1_SUB_ROWS, 128), jnp.int32),
        pltpu.VMEM((S1_SUB_ROWS, 128), jnp.float32),
        pltpu.VMEM((128,), jnp.float32),
        pltpu.VMEM((B,), jnp.float32),
        pltpu.VMEM_SHARED((B,), jnp.float32),
        pltpu.VMEM_SHARED((B,), jnp.float32),
    ],
    compiler_params=_sc_params,
)
def _s1(ids2d, p2d, zpart, cpart, idx_s, val_s, ones_s, zero_s, zsh, csh):
    c = lax.axis_index("c")
    s = lax.axis_index("s")
    wid = s * NC + c

    _fill(ones_s, 128, 1.0)

    @pl.when(s == 0)
    def _():
        _fill(zero_s, B, 0.0)
        pltpu.sync_copy(zero_s, zsh)
        pltpu.sync_copy(zero_s, csh)

    plsc.subcore_barrier()

    for sub in range(ROWS_PER // S1_SUB_ROWS):
        row0 = wid * ROWS_PER + sub * S1_SUB_ROWS
        pltpu.sync_copy(ids2d.at[pl.ds(row0, S1_SUB_ROWS)], idx_s)
        pltpu.sync_copy(p2d.at[pl.ds(row0, S1_SUB_ROWS)], val_s)

        def body(j, _):
            pltpu.sync_copy(val_s.at[j], zsh.at[idx_s.at[j]], add=True)
            pltpu.sync_copy(ones_s, csh.at[idx_s.at[j]], add=True)
            return 0

        lax.fori_loop(0, S1_SUB_ROWS, body, 0)

    plsc.subcore_barrier()
    sl = B // NS
    pltpu.sync_copy(zsh.at[pl.ds(s * sl, sl)], zpart.at[c, pl.ds(s * sl, sl)])
    pltpu.sync_copy(csh.at[pl.ds(s * sl, sl)], cpart.at[c, pl.ds(s * sl, sl)])


@functools.partial(
    pl.kernel,
    mesh=_mesh,
    out_type=[jax.ShapeDtypeStruct((NW, B), jnp.float32),
              jax.ShapeDtypeStruct((NW, B), jnp.float32)],
    scratch_types=[
        pltpu.VMEM((NC, B), jnp.float32),
        pltpu.VMEM((NC, B), jnp.float32),
        pltpu.VMEM((B,), jnp.float32),
        pltpu.VMEM((B,), jnp.float32),
        pltpu.VMEM((B,), jnp.float32),
        pltpu.VMEM((B,), jnp.float32),
        pltpu.VMEM((B,), jnp.float32),
        pltpu.VMEM((S2_SUB,), jnp.int32),
        pltpu.VMEM((S2_SUB,), jnp.float32),
        pltpu.VMEM((S2_SUB,), jnp.float32),
    ],
    compiler_params=_sc_params,
)
def _s2(ids1d, p1d, w1d, zpart, cpart, stop, bestv, bestq,
        zc_s, cc_s, ss_s, dinv_s, it_s, mv_s, mq_s, ids_s, p_s, w_s):
    c = lax.axis_index("c")
    s = lax.axis_index("s")
    wid = s * NC + c

    pltpu.sync_copy(zpart, zc_s)
    pltpu.sync_copy(cpart, cc_s)
    pltpu.sync_copy(stop, ss_s)

    def pro_body(i, _):
        ds = pl.ds(i * L, L)
        z = zc_s[0, ds] + zc_s[1, ds]
        cnt = cc_s[0, ds] + cc_s[1, ds]
        d = z + jnp.exp(ss_s[ds])
        dinv_s[ds] = 1.0 / d
        it_s[ds] = 1.0 / (cnt + 1.0)
        zero = jnp.zeros((L,), jnp.float32)
        mv_s[ds] = zero
        mq_s[ds] = zero
        return 0

    lax.fori_loop(0, B // L, pro_body, 0)

    iota = lax.broadcasted_iota(jnp.int32, (L,), 0)
    nxt_idx = jnp.minimum(iota + 1, L - 1)

    for sub in range(C_PER // S2_SUB):
        base = wid * C_PER + sub * S2_SUB
        pltpu.sync_copy(ids1d.at[pl.ds(base, S2_SUB)], ids_s)
        pltpu.sync_copy(p1d.at[pl.ds(base, S2_SUB)], p_s)
        pltpu.sync_copy(w1d.at[pl.ds(base, S2_SUB)], w_s)

        def body(v, _):
            ds = pl.ds(v * L, L)
            ids16 = ids_s[ds]
            pv = p_s[ds]
            wv = w_s[ds]
            dg = plsc.load_gather(dinv_s, [ids16])
            ig = plsc.load_gather(it_s, [ids16])
            q = (1.0 - RAP) * pv * dg + RAP * ig
            cur_s = q * wv
            cur_q = q
            for d in (1, 2, 4, 8):
                idxs = jnp.maximum(iota - d, 0)
                sh_s = cur_s.at[idxs].get(mode="promise_in_bounds")
                sh_q = cur_q.at[idxs].get(mode="promise_in_bounds")
                sh_id = ids16.at[idxs].get(mode="promise_in_bounds")
                same = sh_id == ids16
                cur_q = jnp.where(same & (sh_s >= cur_s), sh_q, cur_q)
                cur_s = jnp.where(same, jnp.maximum(sh_s, cur_s), cur_s)
            nxt_id = ids16.at[nxt_idx].get(mode="promise_in_bounds")
            is_last = (ids16 != nxt_id) | (iota == L - 1)
            mv = plsc.load_gather(mv_s, [ids16])
            mq = plsc.load_gather(mq_s, [ids16])
            upd = cur_s > mv
            plsc.store_scatter(mv_s, [ids16], jnp.where(upd, cur_s, mv),
                               mask=is_last)
            plsc.store_scatter(mq_s, [ids16], jnp.where(upd, cur_q, mq),
                               mask=is_last)
            return 0

        lax.fori_loop(0, S2_SUB // L, body, 0)

    pltpu.sync_copy(mv_s, bestv.at[wid])
    pltpu.sync_copy(mq_s, bestq.at[wid])


def _e3_body(bestv_ref, bestq_ref, zpart_ref, cpart_ref, stop_ref, out_ref):
    best = bestv_ref[0]
    q = bestq_ref[0]
    for w in range(1, NW):
        v = bestv_ref[w]
        upd = v > best
        best = jnp.where(upd, v, best)
        q = jnp.where(upd, bestq_ref[w], q)
    z = zpart_ref[0] + zpart_ref[1]
    cnt = cpart_ref[0] + cpart_ref[1]
    exp_stop = jnp.exp(stop_ref[...])
    dinv = 1.0 / (z + exp_stop)
    it = 1.0 / (cnt + 1.0)
    final_stop = (1.0 - RAP) * exp_stop * dinv + RAP * it
    log_stop = jnp.log(jnp.clip(final_stop, PROB_EPS, None))
    log_edge = jnp.log(jnp.clip(q, PROB_EPS, None))
    out_ref[...] = jnp.where(final_stop >= best, log_stop, log_edge)


def _e3(bestv, bestq, zpart, cpart, stop_logits):
    rb = B // 128
    out = pl.pallas_call(
        _e3_body,
        out_shape=jax.ShapeDtypeStruct((rb, 128), jnp.float32),
    )(bestv.reshape(NW, rb, 128), bestq.reshape(NW, rb, 128),
      zpart.reshape(NC, rb, 128), cpart.reshape(NC, rb, 128),
      stop_logits.reshape(rb, 128))
    return out.reshape(B)


def kernel(edge_logits, stop_logits, edge_scores, u, edge_batch):
    p2, w2 = _e1(edge_logits, edge_scores, u)
    ids2d = edge_batch.reshape(E // 128, 128)
    zpart, cpart = _s1(ids2d, p2)
    bestv, bestq = _s2(edge_batch, p2.reshape(E), w2.reshape(E),
                       zpart, cpart, stop_logits)
    return _e3(bestv, bestq, zpart, cpart, stop_logits)


# speedup vs baseline: 215.7291x; 1.0763x over previous
import functools

import jax
import jax.numpy as jnp
from jax import lax
from jax.experimental import pallas as pl
from jax.experimental.pallas import tpu as pltpu
from jax.experimental.pallas import tpu_sc as plsc

RAP = 0.1
PRIOR_EPS = 1e-4
PROB_EPS = 1e-12

E = 1048576
B = 4096
NC, NS, L = 2, 16, 16
NW = NC * NS
C_PER = E // NW
ROWS_PER = C_PER // 128
S1_SUB_ROWS = 64
S2_SUB = 16384
_mesh = plsc.VectorSubcoreMesh(core_axis_name="c", subcore_axis_name="s")
_sc_params = pltpu.CompilerParams(needs_layout_passes=False)


def _e1_body(el_ref, es_ref, u_ref, p_ref, w_ref):
    p_ref[...] = jnp.exp(el_ref[...]) * jnp.sqrt(
        jnp.clip(es_ref[...], PRIOR_EPS, None))
    uc = jnp.clip(u_ref[...], 1e-9, 1.0 - 1e-9)
    w_ref[...] = -1.0 / jnp.log(uc)


def _e1(edge_logits, edge_scores, u):
    rows = E // 128
    grid = 16
    blk = rows // grid
    spec = pl.BlockSpec((blk, 128), lambda i: (i, 0))
    return pl.pallas_call(
        _e1_body,
        grid=(grid,),
        in_specs=[spec] * 3,
        out_specs=[spec] * 2,
        out_shape=[jax.ShapeDtypeStruct((rows, 128), jnp.float32)] * 2,
    )(edge_logits.reshape(rows, 128), edge_scores.reshape(rows, 128),
      u.reshape(rows, 128))


def _fill(ref, n, value):
    v = jnp.full((L,), value, dtype=ref.dtype)

    def body(i, _):
        ref[pl.ds(i * L, L)] = v
        return 0

    lax.fori_loop(0, n // L, body, 0)


@functools.partial(
    pl.kernel,
    mesh=_mesh,
    out_type=[jax.ShapeDtypeStruct((NC, B), jnp.float32),
              jax.ShapeDtypeStruct((NC, B), jnp.float32)],
    scratch_types=[
        pltpu.VMEM((S1_SUB_ROWS, 128), jnp.int32),
        pltpu.VMEM((S1_SUB_ROWS, 128), jnp.float32),
        pltpu.VMEM((128,), jnp.float32),
        pltpu.VMEM((B,), jnp.float32),
        pltpu.VMEM_SHARED((B,), jnp.float32),
        pltpu.VMEM_SHARED((B,), jnp.float32),
        pltpu.SemaphoreType.DMA,
        pltpu.SemaphoreType.DMA,
    ],
    compiler_params=_sc_params,
)
def _s1(ids2d, p2d, zpart, cpart, idx_s, val_s, ones_s, zero_s, zsh, csh,
        zsem, csem):
    c = lax.axis_index("c")
    s = lax.axis_index("s")
    wid = s * NC + c

    _fill(ones_s, 128, 1.0)

    @pl.when(s == 0)
    def _():
        _fill(zero_s, B, 0.0)
        pltpu.sync_copy(zero_s, zsh)
        pltpu.sync_copy(zero_s, csh)

    plsc.subcore_barrier()

    for sub in range(ROWS_PER // S1_SUB_ROWS):
        row0 = wid * ROWS_PER + sub * S1_SUB_ROWS
        pltpu.sync_copy(ids2d.at[pl.ds(row0, S1_SUB_ROWS)], idx_s)
        pltpu.sync_copy(p2d.at[pl.ds(row0, S1_SUB_ROWS)], val_s)

        def fire(j, _):
            pltpu.async_copy(val_s.at[j], zsh.at[idx_s.at[j]], zsem, add=True)
            pltpu.async_copy(ones_s, csh.at[idx_s.at[j]], csem, add=True)
            return 0

        lax.fori_loop(0, S1_SUB_ROWS, fire, 0)

        def drain(j, _):
            pltpu.make_async_copy(val_s.at[j], zsh.at[idx_s.at[j]], zsem).wait()
            pltpu.make_async_copy(ones_s, csh.at[idx_s.at[j]], csem).wait()
            return 0

        lax.fori_loop(0, S1_SUB_ROWS, drain, 0)

    plsc.subcore_barrier()
    sl = B // NS
    pltpu.sync_copy(zsh.at[pl.ds(s * sl, sl)], zpart.at[c, pl.ds(s * sl, sl)])
    pltpu.sync_copy(csh.at[pl.ds(s * sl, sl)], cpart.at[c, pl.ds(s * sl, sl)])


@functools.partial(
    pl.kernel,
    mesh=_mesh,
    out_type=[jax.ShapeDtypeStruct((NW, B), jnp.float32),
              jax.ShapeDtypeStruct((NW, B), jnp.float32)],
    scratch_types=[
        pltpu.VMEM((NC, B), jnp.float32),
        pltpu.VMEM((NC, B), jnp.float32),
        pltpu.VMEM((B,), jnp.float32),
        pltpu.VMEM((B,), jnp.float32),
        pltpu.VMEM((B,), jnp.float32),
        pltpu.VMEM((B,), jnp.float32),
        pltpu.VMEM((B,), jnp.float32),
        pltpu.VMEM((S2_SUB,), jnp.int32),
        pltpu.VMEM((S2_SUB,), jnp.float32),
        pltpu.VMEM((S2_SUB,), jnp.float32),
    ],
    compiler_params=_sc_params,
)
def _s2(ids1d, p1d, w1d, zpart, cpart, stop, bestv, bestq,
        zc_s, cc_s, ss_s, dinv_s, it_s, mv_s, mq_s, ids_s, p_s, w_s):
    c = lax.axis_index("c")
    s = lax.axis_index("s")
    wid = s * NC + c

    pltpu.sync_copy(zpart, zc_s)
    pltpu.sync_copy(cpart, cc_s)
    pltpu.sync_copy(stop, ss_s)

    def pro_body(i, _):
        ds = pl.ds(i * L, L)
        z = zc_s[0, ds] + zc_s[1, ds]
        cnt = cc_s[0, ds] + cc_s[1, ds]
        d = z + jnp.exp(ss_s[ds])
        dinv_s[ds] = 1.0 / d
        it_s[ds] = 1.0 / (cnt + 1.0)
        zero = jnp.zeros((L,), jnp.float32)
        mv_s[ds] = zero
        mq_s[ds] = zero
        return 0

    lax.fori_loop(0, B // L, pro_body, 0)

    iota = lax.broadcasted_iota(jnp.int32, (L,), 0)
    nxt_idx = jnp.minimum(iota + 1, L - 1)

    for sub in range(C_PER // S2_SUB):
        base = wid * C_PER + sub * S2_SUB
        pltpu.sync_copy(ids1d.at[pl.ds(base, S2_SUB)], ids_s)
        pltpu.sync_copy(p1d.at[pl.ds(base, S2_SUB)], p_s)
        pltpu.sync_copy(w1d.at[pl.ds(base, S2_SUB)], w_s)

        def body(v, _):
            ds = pl.ds(v * L, L)
            ids16 = ids_s[ds]
            pv = p_s[ds]
            wv = w_s[ds]
            dg = plsc.load_gather(dinv_s, [ids16])
            ig = plsc.load_gather(it_s, [ids16])
            q = (1.0 - RAP) * pv * dg + RAP * ig
            cur_s = q * wv
            cur_q = q
            for d in (1, 2, 4, 8):
                idxs = jnp.maximum(iota - d, 0)
                sh_s = cur_s.at[idxs].get(mode="promise_in_bounds")
                sh_q = cur_q.at[idxs].get(mode="promise_in_bounds")
                sh_id = ids16.at[idxs].get(mode="promise_in_bounds")
                same = sh_id == ids16
                cur_q = jnp.where(same & (sh_s >= cur_s), sh_q, cur_q)
                cur_s = jnp.where(same, jnp.maximum(sh_s, cur_s), cur_s)
            nxt_id = ids16.at[nxt_idx].get(mode="promise_in_bounds")
            is_last = (ids16 != nxt_id) | (iota == L - 1)
            mv = plsc.load_gather(mv_s, [ids16])
            mq = plsc.load_gather(mq_s, [ids16])
            upd = cur_s > mv
            plsc.store_scatter(mv_s, [ids16], jnp.where(upd, cur_s, mv),
                               mask=is_last)
            plsc.store_scatter(mq_s, [ids16], jnp.where(upd, cur_q, mq),
                               mask=is_last)
            return 0

        lax.fori_loop(0, S2_SUB // L, body, 0)

    pltpu.sync_copy(mv_s, bestv.at[wid])
    pltpu.sync_copy(mq_s, bestq.at[wid])


def _e3_body(bestv_ref, bestq_ref, zpart_ref, cpart_ref, stop_ref, out_ref):
    best = bestv_ref[0]
    q = bestq_ref[0]
    for w in range(1, NW):
        v = bestv_ref[w]
        upd = v > best
        best = jnp.where(upd, v, best)
        q = jnp.where(upd, bestq_ref[w], q)
    z = zpart_ref[0] + zpart_ref[1]
    cnt = cpart_ref[0] + cpart_ref[1]
    exp_stop = jnp.exp(stop_ref[...])
    dinv = 1.0 / (z + exp_stop)
    it = 1.0 / (cnt + 1.0)
    final_stop = (1.0 - RAP) * exp_stop * dinv + RAP * it
    log_stop = jnp.log(jnp.clip(final_stop, PROB_EPS, None))
    log_edge = jnp.log(jnp.clip(q, PROB_EPS, None))
    out_ref[...] = jnp.where(final_stop >= best, log_stop, log_edge)


def _e3(bestv, bestq, zpart, cpart, stop_logits):
    rb = B // 128
    out = pl.pallas_call(
        _e3_body,
        out_shape=jax.ShapeDtypeStruct((rb, 128), jnp.float32),
    )(bestv.reshape(NW, rb, 128), bestq.reshape(NW, rb, 128),
      zpart.reshape(NC, rb, 128), cpart.reshape(NC, rb, 128),
      stop_logits.reshape(rb, 128))
    return out.reshape(B)


def kernel(edge_logits, stop_logits, edge_scores, u, edge_batch):
    p2, w2 = _e1(edge_logits, edge_scores, u)
    ids2d = edge_batch.reshape(E // 128, 128)
    zpart, cpart = _s1(ids2d, p2)
    bestv, bestq = _s2(edge_batch, p2.reshape(E), w2.reshape(E),
                       zpart, cpart, stop_logits)
    return _e3(bestv, bestq, zpart, cpart, stop_logits)


# speedup vs baseline: 371.9957x; 1.7244x over previous
import functools

import jax
import jax.numpy as jnp
from jax import lax
from jax.experimental import pallas as pl
from jax.experimental.pallas import tpu as pltpu
from jax.experimental.pallas import tpu_sc as plsc

RAP = 0.1
PRIOR_EPS = 1e-4
PROB_EPS = 1e-12

E = 1048576
B = 4096
NC, NS, L = 2, 16, 16
NW = NC * NS
C_PER = E // NW
ROWS_PER = C_PER // 128
S1_SUB_ROWS = 64
S2_SUB = 16384
_mesh = plsc.VectorSubcoreMesh(core_axis_name="c", subcore_axis_name="s")
_sc_params = pltpu.CompilerParams(needs_layout_passes=False)


def _e1_body(el_ref, es_ref, u_ref, p_ref, w_ref):
    p_ref[...] = jnp.exp(el_ref[...]) * jnp.sqrt(
        jnp.clip(es_ref[...], PRIOR_EPS, None))
    uc = jnp.clip(u_ref[...], 1e-9, 1.0 - 1e-9)
    w_ref[...] = -1.0 / jnp.log(uc)


def _e1(edge_logits, edge_scores, u):
    rows = E // 128
    grid = 16
    blk = rows // grid
    spec = pl.BlockSpec((blk, 128), lambda i: (i, 0))
    return pl.pallas_call(
        _e1_body,
        grid=(grid,),
        in_specs=[spec] * 3,
        out_specs=[spec] * 2,
        out_shape=[jax.ShapeDtypeStruct((rows, 128), jnp.float32)] * 2,
    )(edge_logits.reshape(rows, 128), edge_scores.reshape(rows, 128),
      u.reshape(rows, 128))


def _fill(ref, n, value):
    v = jnp.full((L,), value, dtype=ref.dtype)

    def body(i, _):
        ref[pl.ds(i * L, L)] = v
        return 0

    lax.fori_loop(0, n // L, body, 0)


SL = B // NS


@functools.partial(
    pl.kernel,
    mesh=_mesh,
    out_type=[jax.ShapeDtypeStruct((NC, B), jnp.float32),
              jax.ShapeDtypeStruct((NC, B), jnp.float32)],
    scratch_types=[
        pltpu.VMEM((C_PER,), jnp.int32),
        pltpu.VMEM((C_PER,), jnp.float32),
        pltpu.VMEM((B,), jnp.float32),
        pltpu.VMEM((B,), jnp.float32),
        pltpu.VMEM((NS, SL), jnp.float32),
        pltpu.VMEM((NS, SL), jnp.float32),
        pltpu.VMEM((SL,), jnp.float32),
        pltpu.VMEM((SL,), jnp.float32),
        pltpu.VMEM_SHARED((NS, NS, SL), jnp.float32),
        pltpu.VMEM_SHARED((NS, NS, SL), jnp.float32),
        pltpu.SemaphoreType.DMA,
        pltpu.SemaphoreType.DMA,
    ],
    compiler_params=_sc_params,
)
def _s1(ids1d, p1d, zpart, cpart, ids_s, p_s, zloc, cloc, mz, mc,
        zred, cred, zsl, csl, sem0, sem1):
    c = lax.axis_index("c")
    s = lax.axis_index("s")
    wid = s * NC + c
    base = wid * C_PER

    cp0 = pltpu.async_copy(ids1d.at[pl.ds(base, C_PER)], ids_s, sem0)
    cp1 = pltpu.async_copy(p1d.at[pl.ds(base, C_PER)], p_s, sem1)
    _fill(zloc, B, 0.0)
    _fill(cloc, B, 0.0)
    cp0.wait()
    cp1.wait()

    iota = lax.broadcasted_iota(jnp.int32, (L,), 0)
    nxt_idx = jnp.minimum(iota + 1, L - 1)
    ones = jnp.ones((L,), jnp.float32)

    def body(v, _):
        ds = pl.ds(v * L, L)
        ids16 = ids_s[ds]
        sp = p_s[ds]
        sc = ones
        for d in (1, 2, 4, 8):
            idxs = jnp.maximum(iota - d, 0)
            sh_id = ids16.at[idxs].get(mode="promise_in_bounds")
            sh_p = sp.at[idxs].get(mode="promise_in_bounds")
            sh_c = sc.at[idxs].get(mode="promise_in_bounds")
            ok = (sh_id == ids16) & (iota >= d)
            sp = jnp.where(ok, sp + sh_p, sp)
            sc = jnp.where(ok, sc + sh_c, sc)
        nxt_id = ids16.at[nxt_idx].get(mode="promise_in_bounds")
        is_last = (ids16 != nxt_id) | (iota == L - 1)
        plsc.addupdate_scatter(zloc, [ids16], sp, mask=is_last)
        plsc.addupdate_scatter(cloc, [ids16], sc, mask=is_last)
        return 0

    lax.fori_loop(0, C_PER // L, body, 0)

    for w in range(NS):
        pltpu.sync_copy(zloc.at[pl.ds(w * SL, SL)], zsl.at[w, s])
        pltpu.sync_copy(cloc.at[pl.ds(w * SL, SL)], csl.at[w, s])
    plsc.subcore_barrier()
    pltpu.sync_copy(zsl.at[s], mz)
    pltpu.sync_copy(csl.at[s], mc)

    def red(i, _):
        ds = pl.ds(i * L, L)
        az = mz[0, ds]
        ac = mc[0, ds]
        for r in range(1, NS):
            az = az + mz[r, ds]
            ac = ac + mc[r, ds]
        zred[ds] = az
        cred[ds] = ac
        return 0

    lax.fori_loop(0, SL // L, red, 0)
    pltpu.sync_copy(zred, zpart.at[c, pl.ds(s * SL, SL)])
    pltpu.sync_copy(cred, cpart.at[c, pl.ds(s * SL, SL)])


@functools.partial(
    pl.kernel,
    mesh=_mesh,
    out_type=[jax.ShapeDtypeStruct((NW, B), jnp.float32),
              jax.ShapeDtypeStruct((NW, B), jnp.float32)],
    scratch_types=[
        pltpu.VMEM((NC, B), jnp.float32),
        pltpu.VMEM((NC, B), jnp.float32),
        pltpu.VMEM((B,), jnp.float32),
        pltpu.VMEM((B,), jnp.float32),
        pltpu.VMEM((B,), jnp.float32),
        pltpu.VMEM((B,), jnp.float32),
        pltpu.VMEM((B,), jnp.float32),
        pltpu.VMEM((S2_SUB,), jnp.int32),
        pltpu.VMEM((S2_SUB,), jnp.float32),
        pltpu.VMEM((S2_SUB,), jnp.float32),
    ],
    compiler_params=_sc_params,
)
def _s2(ids1d, p1d, w1d, zpart, cpart, stop, bestv, bestq,
        zc_s, cc_s, ss_s, dinv_s, it_s, mv_s, mq_s, ids_s, p_s, w_s):
    c = lax.axis_index("c")
    s = lax.axis_index("s")
    wid = s * NC + c

    pltpu.sync_copy(zpart, zc_s)
    pltpu.sync_copy(cpart, cc_s)
    pltpu.sync_copy(stop, ss_s)

    def pro_body(i, _):
        ds = pl.ds(i * L, L)
        z = zc_s[0, ds] + zc_s[1, ds]
        cnt = cc_s[0, ds] + cc_s[1, ds]
        d = z + jnp.exp(ss_s[ds])
        dinv_s[ds] = 1.0 / d
        it_s[ds] = 1.0 / (cnt + 1.0)
        zero = jnp.zeros((L,), jnp.float32)
        mv_s[ds] = zero
        mq_s[ds] = zero
        return 0

    lax.fori_loop(0, B // L, pro_body, 0)

    iota = lax.broadcasted_iota(jnp.int32, (L,), 0)
    nxt_idx = jnp.minimum(iota + 1, L - 1)

    for sub in range(C_PER // S2_SUB):
        base = wid * C_PER + sub * S2_SUB
        pltpu.sync_copy(ids1d.at[pl.ds(base, S2_SUB)], ids_s)
        pltpu.sync_copy(p1d.at[pl.ds(base, S2_SUB)], p_s)
        pltpu.sync_copy(w1d.at[pl.ds(base, S2_SUB)], w_s)

        def body(v, _):
            ds = pl.ds(v * L, L)
            ids16 = ids_s[ds]
            pv = p_s[ds]
            wv = w_s[ds]
            dg = plsc.load_gather(dinv_s, [ids16])
            ig = plsc.load_gather(it_s, [ids16])
            q = (1.0 - RAP) * pv * dg + RAP * ig
            cur_s = q * wv
            cur_q = q
            for d in (1, 2, 4, 8):
                idxs = jnp.maximum(iota - d, 0)
                sh_s = cur_s.at[idxs].get(mode="promise_in_bounds")
                sh_q = cur_q.at[idxs].get(mode="promise_in_bounds")
                sh_id = ids16.at[idxs].get(mode="promise_in_bounds")
                same = sh_id == ids16
                cur_q = jnp.where(same & (sh_s >= cur_s), sh_q, cur_q)
                cur_s = jnp.where(same, jnp.maximum(sh_s, cur_s), cur_s)
            nxt_id = ids16.at[nxt_idx].get(mode="promise_in_bounds")
            is_last = (ids16 != nxt_id) | (iota == L - 1)
            mv = plsc.load_gather(mv_s, [ids16])
            mq = plsc.load_gather(mq_s, [ids16])
            upd = cur_s > mv
            plsc.store_scatter(mv_s, [ids16], jnp.where(upd, cur_s, mv),
                               mask=is_last)
            plsc.store_scatter(mq_s, [ids16], jnp.where(upd, cur_q, mq),
                               mask=is_last)
            return 0

        lax.fori_loop(0, S2_SUB // L, body, 0)

    pltpu.sync_copy(mv_s, bestv.at[wid])
    pltpu.sync_copy(mq_s, bestq.at[wid])


def _e3_body(bestv_ref, bestq_ref, zpart_ref, cpart_ref, stop_ref, out_ref):
    best = bestv_ref[0]
    q = bestq_ref[0]
    for w in range(1, NW):
        v = bestv_ref[w]
        upd = v > best
        best = jnp.where(upd, v, best)
        q = jnp.where(upd, bestq_ref[w], q)
    z = zpart_ref[0] + zpart_ref[1]
    cnt = cpart_ref[0] + cpart_ref[1]
    exp_stop = jnp.exp(stop_ref[...])
    dinv = 1.0 / (z + exp_stop)
    it = 1.0 / (cnt + 1.0)
    final_stop = (1.0 - RAP) * exp_stop * dinv + RAP * it
    log_stop = jnp.log(jnp.clip(final_stop, PROB_EPS, None))
    log_edge = jnp.log(jnp.clip(q, PROB_EPS, None))
    out_ref[...] = jnp.where(final_stop >= best, log_stop, log_edge)


def _e3(bestv, bestq, zpart, cpart, stop_logits):
    rb = B // 128
    out = pl.pallas_call(
        _e3_body,
        out_shape=jax.ShapeDtypeStruct((rb, 128), jnp.float32),
    )(bestv.reshape(NW, rb, 128), bestq.reshape(NW, rb, 128),
      zpart.reshape(NC, rb, 128), cpart.reshape(NC, rb, 128),
      stop_logits.reshape(rb, 128))
    return out.reshape(B)


def kernel(edge_logits, stop_logits, edge_scores, u, edge_batch):
    p2, w2 = _e1(edge_logits, edge_scores, u)
    zpart, cpart = _s1(edge_batch, p2.reshape(E))
    bestv, bestq = _s2(edge_batch, p2.reshape(E), w2.reshape(E),
                       zpart, cpart, stop_logits)
    return _e3(bestv, bestq, zpart, cpart, stop_logits)
